# Initial kernel scaffold; baseline (speedup 1.0000x reference)
import jax
import jax.numpy as jnp
from jax.experimental import pallas as pl


def kernel(x, h, c, edge_index, edge_weight, W_i, b_i, W_f, b_f, W_o, b_o, W_g, b_g):
    raise NotImplementedError("write your pallas kernel here")


# speedup vs baseline: 9.7690x; 9.7690x over previous
import functools

import jax
import jax.numpy as jnp
from jax import lax
from jax.experimental import pallas as pl
from jax.experimental.pallas import tpu as pltpu
from jax.experimental.pallas import tpu_sc as plsc

N = 10000
E = 320000
CH = 128
CHK = 64
NQ = 8
K = 80
NC, NS = 2, 16
ROWS = E // K
ROWS_A = ROWS // (NC * NS)
ROWS_C = ROWS // NS
PASSES = NQ // NC

_MESH = plsc.VectorSubcoreMesh(
    core_axis_name="c", subcore_axis_name="s", num_cores=NC, num_subcores=NS)
_SC_PARAMS = pltpu.CompilerParams(use_tc_tiling_on_sc=False)


@functools.partial(
    pl.kernel,
    out_type=jax.ShapeDtypeStruct((NC, 1, N), jnp.float32),
    mesh=_MESH,
    scratch_types=[
        pltpu.VMEM((ROWS_A, K), jnp.int32),
        pltpu.VMEM((ROWS_A, K), jnp.float32),
        pltpu.VMEM_SHARED((1, N), jnp.float32),
    ],
    compiler_params=_SC_PARAMS,
)
def _deg_kernel(dst3d, ew3d, zeros_n, degp, idx_v, ew_v, deg_sh):
    c = lax.axis_index("c")
    s = lax.axis_index("s")
    wid = s * NC + c

    @pl.when(s == 0)
    def _():
        pltpu.sync_copy(zeros_n, deg_sh)

    pltpu.sync_copy(dst3d.at[wid], idx_v)
    pltpu.sync_copy(ew3d.at[wid], ew_v)
    plsc.subcore_barrier()

    def body(b, carry):
        pltpu.sync_copy(ew_v.at[b], deg_sh.at[0].at[idx_v.at[b]], add=True)
        return carry

    lax.fori_loop(0, ROWS_A, body, 0)
    plsc.subcore_barrier()

    @pl.when(s == 0)
    def _():
        pltpu.sync_copy(deg_sh, degp.at[c])


def _dis_body(degp_ref, dis_ref):
    deg = degp_ref[0, 0, :] + degp_ref[1, 0, :] + 1.0
    dis = jnp.where(deg > 0, lax.rsqrt(jnp.maximum(deg, 1e-12)), 0.0)
    dis_ref[...] = dis[:, None]


_diskern = pl.pallas_call(
    _dis_body,
    in_specs=[pl.BlockSpec((2, 1, N), lambda: (0, 0, 0))],
    out_specs=pl.BlockSpec((N, 1), lambda: (0, 0)),
    out_shape=jax.ShapeDtypeStruct((N, 1), jnp.float32),
)


def _scale_body(dis_ref, x0, h0, x1, h1, out_ref):
    disc = dis_ref[...]
    srcs = (x0, h0, x1, h1)
    for qi in range(NQ):
        half = pl.ds((qi % 2) * CHK, CHK)
        out_ref[qi] = srcs[qi // 2][:, half] * disc


_BN = 2000
_xscale = pl.pallas_call(
    _scale_body,
    grid=(N // _BN,),
    in_specs=[
        pl.BlockSpec((_BN, 1), lambda i: (i, 0)),
        pl.BlockSpec((_BN, CH), lambda i: (i, 0)),
        pl.BlockSpec((_BN, CH), lambda i: (i, 0)),
        pl.BlockSpec((_BN, CH), lambda i: (i, 0)),
        pl.BlockSpec((_BN, CH), lambda i: (i, 0)),
    ],
    out_specs=pl.BlockSpec((NQ, _BN, CHK), lambda i: (0, i, 0)),
    out_shape=jax.ShapeDtypeStruct((NQ, N, CHK), jnp.float32),
)


_SPLIT = 640
_LAST = N - (NS - 1) * _SPLIT


def _node_copy(s, mk_src, mk_dst):
    @pl.when(s < NS - 1)
    def _():
        off = pl.multiple_of(s * _SPLIT, _SPLIT)
        pltpu.sync_copy(mk_src(off, _SPLIT), mk_dst(off, _SPLIT))

    @pl.when(s == NS - 1)
    def _():
        off = (NS - 1) * _SPLIT
        pltpu.sync_copy(mk_src(off, _LAST), mk_dst(off, _LAST))


@functools.partial(
    pl.kernel,
    out_type=jax.ShapeDtypeStruct((NQ, N, CHK), jnp.float32),
    mesh=_MESH,
    scratch_types=[
        pltpu.VMEM((ROWS_C, K), jnp.int32),
        pltpu.VMEM((ROWS_C, K), jnp.int32),
        pltpu.VMEM((K, 16), jnp.float32),
        pltpu.VMEM((K, CHK), jnp.float32),
        pltpu.VMEM_SHARED((N, CHK), jnp.float32),
    ],
    compiler_params=_SC_PARAMS,
)
def _prop_kernel(src3d, dst3d, ew3d16, xhs, aggp,
                 src_v, dst_v, w_v, rows_v, agg_sh):
    c = lax.axis_index("c")
    s = lax.axis_index("s")
    ebase = s * ROWS_C
    pltpu.sync_copy(src3d.at[s], src_v)
    pltpu.sync_copy(dst3d.at[s], dst_v)

    for p in range(PASSES):
        q = PASSES * c + p
        _node_copy(s,
                   lambda off, n: xhs.at[q].at[pl.ds(off, n)],
                   lambda off, n: agg_sh.at[pl.ds(off, n)])
        plsc.subcore_barrier()

        def body(b, carry):
            pltpu.sync_copy(xhs.at[q].at[src_v.at[b]], rows_v)
            pltpu.sync_copy(ew3d16.at[ebase + b], w_v)

            for e in range(K):
                w = w_v[e]
                for j in range(CHK // 16):
                    sl = pl.ds(j * 16, 16)
                    rows_v[e, sl] = rows_v[e, sl] * w
            pltpu.sync_copy(rows_v, agg_sh.at[dst_v.at[b]], add=True)
            return carry

        lax.fori_loop(0, ROWS_C, body, 0)
        plsc.subcore_barrier()
        _node_copy(s,
                   lambda off, n: agg_sh.at[pl.ds(off, n)],
                   lambda off, n: aggp.at[q].at[pl.ds(off, n)])
        plsc.subcore_barrier()


def _gate_body(dis_ref, aggp_ref, c_ref, w4_ref, b4_ref, h_out, c_out):
    dis = dis_ref[...]
    w4 = w4_ref[...]
    b4 = b4_ref[...]
    for b in range(2):
        agg = jnp.concatenate(
            [aggp_ref[4 * b + j] for j in range(4)], axis=1) * dis
        z = jnp.dot(agg, w4, preferred_element_type=jnp.float32) + b4
        i = jax.nn.sigmoid(z[:, :CH])
        f = jax.nn.sigmoid(z[:, CH:2 * CH])
        o = jax.nn.sigmoid(z[:, 2 * CH:3 * CH])
        g = jnp.tanh(z[:, 3 * CH:])
        cn = f * c_ref[b] + i * g
        h_out[b] = o * jnp.tanh(cn)
        c_out[b] = cn


_BG = 1000
_gates = pl.pallas_call(
    _gate_body,
    grid=(N // _BG,),
    in_specs=[
        pl.BlockSpec((_BG, 1), lambda i: (i, 0)),
        pl.BlockSpec((NQ, _BG, CHK), lambda i: (0, i, 0)),
        pl.BlockSpec((2, _BG, CH), lambda i: (0, i, 0)),
        pl.BlockSpec((2 * CH, 4 * CH), lambda i: (0, 0)),
        pl.BlockSpec((1, 4 * CH), lambda i: (0, 0)),
    ],
    out_specs=[
        pl.BlockSpec((2, _BG, CH), lambda i: (0, i, 0)),
        pl.BlockSpec((2, _BG, CH), lambda i: (0, i, 0)),
    ],
    out_shape=[
        jax.ShapeDtypeStruct((2, N, CH), jnp.float32),
        jax.ShapeDtypeStruct((2, N, CH), jnp.float32),
    ],
)


def kernel(x, h, c, edge_index, edge_weight, W_i, b_i, W_f, b_f, W_o, b_o, W_g, b_g):
    dst_a = edge_index[1].reshape(NC * NS, ROWS_A, K)
    ew_a = edge_weight.reshape(NC * NS, ROWS_A, K)
    src_c = edge_index[0].reshape(NS, ROWS_C, K)
    dst_c = edge_index[1].reshape(NS, ROWS_C, K)
    zeros_n = jnp.zeros((1, N), jnp.float32)
    degp = _deg_kernel(dst_a, ew_a, zeros_n)
    dis2d = _diskern(degp)
    xhs = _xscale(dis2d, x[0], h[0], x[1], h[1])
    ew16 = edge_weight.reshape(ROWS, K)
    ew3d16 = jnp.broadcast_to(ew16[:, :, None], (ROWS, K, 16))
    aggp = _prop_kernel(src_c, dst_c, ew3d16, xhs)
    w4 = jnp.concatenate([W_i, W_f, W_o, W_g], axis=1)
    b4 = jnp.concatenate([b_i, b_f, b_o, b_g]).reshape(1, 4 * CH)
    h_n, c_n = _gates(dis2d, aggp, c, w4, b4)
    return h_n, c_n


# speedup vs baseline: 20.6846x; 2.1174x over previous
import functools

import jax
import jax.numpy as jnp
from jax import lax
from jax.experimental import pallas as pl
from jax.experimental.pallas import tpu as pltpu
from jax.experimental.pallas import tpu_sc as plsc

N = 10000
E = 320000
CH = 128
CHK = 64
NQ = 8
K = 80
NC, NS = 2, 16
ROWS = E // K
ROWS_A = ROWS // (NC * NS)
ROWS_C = ROWS // NS
PASSES = NQ // NC

_MESH = plsc.VectorSubcoreMesh(
    core_axis_name="c", subcore_axis_name="s", num_cores=NC, num_subcores=NS)
_SC_PARAMS = pltpu.CompilerParams(use_tc_tiling_on_sc=False)


@functools.partial(
    pl.kernel,
    out_type=jax.ShapeDtypeStruct((NC, 1, N), jnp.float32),
    mesh=_MESH,
    scratch_types=[
        pltpu.VMEM((ROWS_A, K), jnp.int32),
        pltpu.VMEM((ROWS_A, K), jnp.float32),
        pltpu.VMEM_SHARED((1, N), jnp.float32),
    ],
    compiler_params=_SC_PARAMS,
)
def _deg_kernel(dst3d, ew3d, zeros_n, degp, idx_v, ew_v, deg_sh):
    c = lax.axis_index("c")
    s = lax.axis_index("s")
    wid = s * NC + c

    @pl.when(s == 0)
    def _():
        pltpu.sync_copy(zeros_n, deg_sh)

    pltpu.sync_copy(dst3d.at[wid], idx_v)
    pltpu.sync_copy(ew3d.at[wid], ew_v)
    plsc.subcore_barrier()

    def body(b, carry):
        pltpu.sync_copy(ew_v.at[b], deg_sh.at[0].at[idx_v.at[b]], add=True)
        return carry

    lax.fori_loop(0, ROWS_A, body, 0)
    plsc.subcore_barrier()

    @pl.when(s == 0)
    def _():
        pltpu.sync_copy(deg_sh, degp.at[c])


def _dis_body(degp_ref, dis_ref):
    deg = degp_ref[0, 0, :] + degp_ref[1, 0, :] + 1.0
    dis = jnp.where(deg > 0, lax.rsqrt(jnp.maximum(deg, 1e-12)), 0.0)
    dis_ref[...] = dis[:, None]


_diskern = pl.pallas_call(
    _dis_body,
    in_specs=[pl.BlockSpec((2, 1, N), lambda: (0, 0, 0))],
    out_specs=pl.BlockSpec((N, 1), lambda: (0, 0)),
    out_shape=jax.ShapeDtypeStruct((N, 1), jnp.float32),
)


def _scale_body(dis_ref, x0, h0, x1, h1, out_ref):
    disc = dis_ref[...]
    srcs = (x0, h0, x1, h1)
    for qi in range(NQ):
        half = pl.ds((qi % 2) * CHK, CHK)
        out_ref[qi] = srcs[qi // 2][:, half] * disc


_BN = 2000
_xscale = pl.pallas_call(
    _scale_body,
    grid=(N // _BN,),
    in_specs=[
        pl.BlockSpec((_BN, 1), lambda i: (i, 0)),
        pl.BlockSpec((_BN, CH), lambda i: (i, 0)),
        pl.BlockSpec((_BN, CH), lambda i: (i, 0)),
        pl.BlockSpec((_BN, CH), lambda i: (i, 0)),
        pl.BlockSpec((_BN, CH), lambda i: (i, 0)),
    ],
    out_specs=pl.BlockSpec((NQ, _BN, CHK), lambda i: (0, i, 0)),
    out_shape=jax.ShapeDtypeStruct((NQ, N, CHK), jnp.float32),
)


_SPLIT = 640
_LAST = N - (NS - 1) * _SPLIT


def _node_copy(s, mk_src, mk_dst):
    @pl.when(s < NS - 1)
    def _():
        off = pl.multiple_of(s * _SPLIT, _SPLIT)
        pltpu.sync_copy(mk_src(off, _SPLIT), mk_dst(off, _SPLIT))

    @pl.when(s == NS - 1)
    def _():
        off = (NS - 1) * _SPLIT
        pltpu.sync_copy(mk_src(off, _LAST), mk_dst(off, _LAST))


@functools.partial(
    pl.kernel,
    out_type=jax.ShapeDtypeStruct((NQ, N, CHK), jnp.float32),
    mesh=_MESH,
    scratch_types=[
        pltpu.VMEM((ROWS_C, K), jnp.int32),
        pltpu.VMEM((ROWS_C, K), jnp.int32),
        pltpu.VMEM((2, K, 16), jnp.float32),
        pltpu.VMEM((2, K, CHK), jnp.float32),
        pltpu.VMEM((2, K, CHK), jnp.float32),
        pltpu.VMEM_SHARED((N, CHK), jnp.float32),
        pltpu.SemaphoreType.DMA,
        pltpu.SemaphoreType.DMA,
        pltpu.SemaphoreType.DMA,
    ],
    compiler_params=_SC_PARAMS,
)
def _prop_kernel(src3d, dst3d, ew3d16, xhs, aggp,
                 src_v, dst_v, w_v, rows_v, sc_v, agg_sh,
                 gsem, wsem, ssem):
    c = lax.axis_index("c")
    s = lax.axis_index("s")
    ebase = s * ROWS_C
    pltpu.sync_copy(src3d.at[s], src_v)
    pltpu.sync_copy(dst3d.at[s], dst_v)

    def pass_body(p, carry):
        q = PASSES * c + p
        _node_copy(s,
                   lambda off, n: xhs.at[q].at[pl.ds(off, n)],
                   lambda off, n: agg_sh.at[pl.ds(off, n)])
        plsc.subcore_barrier()

        def start_fetch(b):
            buf = b & 1
            pltpu.async_copy(xhs.at[q].at[src_v.at[b]], rows_v.at[buf], gsem)
            pltpu.async_copy(ew3d16.at[ebase + b], w_v.at[buf], wsem)

        start_fetch(0)
        start_fetch(1)

        def body(b, carry2):
            buf = b & 1
            pltpu.make_async_copy(
                xhs.at[q].at[src_v.at[b]], rows_v.at[buf], gsem).wait()
            pltpu.make_async_copy(
                ew3d16.at[ebase + b], w_v.at[buf], wsem).wait()

            @pl.when(b >= 2)
            def _():
                pltpu.make_async_copy(
                    sc_v.at[buf], agg_sh.at[dst_v.at[b - 2]], ssem).wait()

            for e in range(K):
                w = w_v[buf, e]
                for j in range(CHK // 16):
                    sl = pl.ds(j * 16, 16)
                    sc_v[buf, e, sl] = rows_v[buf, e, sl] * w

            pltpu.async_copy(sc_v.at[buf], agg_sh.at[dst_v.at[b]], ssem,
                             add=True)

            @pl.when(b + 2 < ROWS_C)
            def _():
                start_fetch(b + 2)

            return carry2

        lax.fori_loop(0, ROWS_C, body, 0)
        for tail in range(2):
            b = ROWS_C - 2 + tail
            pltpu.make_async_copy(
                sc_v.at[b & 1], agg_sh.at[dst_v.at[b]], ssem).wait()
        plsc.subcore_barrier()
        _node_copy(s,
                   lambda off, n: agg_sh.at[pl.ds(off, n)],
                   lambda off, n: aggp.at[q].at[pl.ds(off, n)])
        plsc.subcore_barrier()
        return carry

    lax.fori_loop(0, PASSES, pass_body, 0)


def _gate_body(dis_ref, aggp_ref, c_ref, w4_ref, b4_ref, h_out, c_out):
    dis = dis_ref[...]
    w4 = w4_ref[...]
    b4 = b4_ref[...]
    for b in range(2):
        agg = jnp.concatenate(
            [aggp_ref[4 * b + j] for j in range(4)], axis=1) * dis
        z = jnp.dot(agg, w4, preferred_element_type=jnp.float32) + b4
        i = jax.nn.sigmoid(z[:, :CH])
        f = jax.nn.sigmoid(z[:, CH:2 * CH])
        o = jax.nn.sigmoid(z[:, 2 * CH:3 * CH])
        g = jnp.tanh(z[:, 3 * CH:])
        cn = f * c_ref[b] + i * g
        h_out[b] = o * jnp.tanh(cn)
        c_out[b] = cn


_BG = 1000
_gates = pl.pallas_call(
    _gate_body,
    grid=(N // _BG,),
    in_specs=[
        pl.BlockSpec((_BG, 1), lambda i: (i, 0)),
        pl.BlockSpec((NQ, _BG, CHK), lambda i: (0, i, 0)),
        pl.BlockSpec((2, _BG, CH), lambda i: (0, i, 0)),
        pl.BlockSpec((2 * CH, 4 * CH), lambda i: (0, 0)),
        pl.BlockSpec((1, 4 * CH), lambda i: (0, 0)),
    ],
    out_specs=[
        pl.BlockSpec((2, _BG, CH), lambda i: (0, i, 0)),
        pl.BlockSpec((2, _BG, CH), lambda i: (0, i, 0)),
    ],
    out_shape=[
        jax.ShapeDtypeStruct((2, N, CH), jnp.float32),
        jax.ShapeDtypeStruct((2, N, CH), jnp.float32),
    ],
)


def kernel(x, h, c, edge_index, edge_weight, W_i, b_i, W_f, b_f, W_o, b_o, W_g, b_g):
    dst_a = edge_index[1].reshape(NC * NS, ROWS_A, K)
    ew_a = edge_weight.reshape(NC * NS, ROWS_A, K)
    src_c = edge_index[0].reshape(NS, ROWS_C, K)
    dst_c = edge_index[1].reshape(NS, ROWS_C, K)
    zeros_n = jnp.zeros((1, N), jnp.float32)
    degp = _deg_kernel(dst_a, ew_a, zeros_n)
    dis2d = _diskern(degp)
    xhs = _xscale(dis2d, x[0], h[0], x[1], h[1])
    ew16 = edge_weight.reshape(ROWS, K)
    ew3d16 = jnp.broadcast_to(ew16[:, :, None], (ROWS, K, 16))
    aggp = _prop_kernel(src_c, dst_c, ew3d16, xhs)
    w4 = jnp.concatenate([W_i, W_f, W_o, W_g], axis=1)
    b4 = jnp.concatenate([b_i, b_f, b_o, b_g]).reshape(1, 4 * CH)
    h_n, c_n = _gates(dis2d, aggp, c, w4, b4)
    return h_n, c_n


# speedup vs baseline: 24.7800x; 1.1980x over previous
import functools

import jax
import jax.numpy as jnp
from jax import lax
from jax.experimental import pallas as pl
from jax.experimental.pallas import tpu as pltpu
from jax.experimental.pallas import tpu_sc as plsc

N = 10000
E = 320000
CH = 128
CHK = 64
NQ = 8
K = 100
NC, NS = 2, 16
ROWS = E // K
ROWS_A = ROWS // (NC * NS)
ROWS_C = ROWS // NS
PASSES = NQ // NC

_MESH = plsc.VectorSubcoreMesh(
    core_axis_name="c", subcore_axis_name="s", num_cores=NC, num_subcores=NS)
_SC_PARAMS = pltpu.CompilerParams(use_tc_tiling_on_sc=False)


@functools.partial(
    pl.kernel,
    out_type=jax.ShapeDtypeStruct((NC, 1, N), jnp.float32),
    mesh=_MESH,
    scratch_types=[
        pltpu.VMEM((ROWS_A, K), jnp.int32),
        pltpu.VMEM((ROWS_A, K), jnp.float32),
        pltpu.VMEM_SHARED((1, N), jnp.float32),
    ],
    compiler_params=_SC_PARAMS,
)
def _deg_kernel(dst3d, ew3d, zeros_n, degp, idx_v, ew_v, deg_sh):
    c = lax.axis_index("c")
    s = lax.axis_index("s")
    wid = s * NC + c

    @pl.when(s == 0)
    def _():
        pltpu.sync_copy(zeros_n, deg_sh)

    pltpu.sync_copy(dst3d.at[wid], idx_v)
    pltpu.sync_copy(ew3d.at[wid], ew_v)
    plsc.subcore_barrier()

    def body(b, carry):
        pltpu.sync_copy(ew_v.at[b], deg_sh.at[0].at[idx_v.at[b]], add=True)
        return carry

    lax.fori_loop(0, ROWS_A, body, 0)
    plsc.subcore_barrier()

    @pl.when(s == 0)
    def _():
        pltpu.sync_copy(deg_sh, degp.at[c])


def _dis_body(degp_ref, dis_ref):
    deg = degp_ref[0, 0, :] + degp_ref[1, 0, :] + 1.0
    dis = jnp.where(deg > 0, lax.rsqrt(jnp.maximum(deg, 1e-12)), 0.0)
    dis_ref[...] = dis[:, None]


_diskern = pl.pallas_call(
    _dis_body,
    in_specs=[pl.BlockSpec((2, 1, N), lambda: (0, 0, 0))],
    out_specs=pl.BlockSpec((N, 1), lambda: (0, 0)),
    out_shape=jax.ShapeDtypeStruct((N, 1), jnp.float32),
)


def _scale_body(dis_ref, x0, h0, x1, h1, out_ref):
    disc = dis_ref[...]
    srcs = (x0, h0, x1, h1)
    for qi in range(NQ):
        half = pl.ds((qi % 2) * CHK, CHK)
        out_ref[qi] = srcs[qi // 2][:, half] * disc


_BN = 2000
_xscale = pl.pallas_call(
    _scale_body,
    grid=(N // _BN,),
    in_specs=[
        pl.BlockSpec((_BN, 1), lambda i: (i, 0)),
        pl.BlockSpec((_BN, CH), lambda i: (i, 0)),
        pl.BlockSpec((_BN, CH), lambda i: (i, 0)),
        pl.BlockSpec((_BN, CH), lambda i: (i, 0)),
        pl.BlockSpec((_BN, CH), lambda i: (i, 0)),
    ],
    out_specs=pl.BlockSpec((NQ, _BN, CHK), lambda i: (0, i, 0)),
    out_shape=jax.ShapeDtypeStruct((NQ, N, CHK), jnp.float32),
)


_SPLIT = 640
_LAST = N - (NS - 1) * _SPLIT


def _node_copy(s, mk_src, mk_dst):
    @pl.when(s < NS - 1)
    def _():
        off = pl.multiple_of(s * _SPLIT, _SPLIT)
        pltpu.sync_copy(mk_src(off, _SPLIT), mk_dst(off, _SPLIT))

    @pl.when(s == NS - 1)
    def _():
        off = (NS - 1) * _SPLIT
        pltpu.sync_copy(mk_src(off, _LAST), mk_dst(off, _LAST))


@functools.partial(
    pl.kernel,
    out_type=jax.ShapeDtypeStruct((NQ, N, CHK), jnp.float32),
    mesh=_MESH,
    scratch_types=[
        pltpu.VMEM((ROWS_C, K), jnp.int32),
        pltpu.VMEM((ROWS_C, K), jnp.int32),
        pltpu.VMEM((3, K, 16), jnp.float32),
        pltpu.VMEM((3, K, CHK), jnp.float32),
        pltpu.VMEM((3, K, CHK), jnp.float32),
        pltpu.VMEM_SHARED((N, CHK), jnp.float32),
        pltpu.SemaphoreType.DMA,
        pltpu.SemaphoreType.DMA,
        pltpu.SemaphoreType.DMA,
    ],
    compiler_params=_SC_PARAMS,
)
def _prop_kernel(src3d, dst3d, ew3d16, xhs, aggp,
                 src_v, dst_v, w_v, rows_v, sc_v, agg_sh,
                 gsem, wsem, ssem):
    c = lax.axis_index("c")
    s = lax.axis_index("s")
    ebase = s * ROWS_C
    pltpu.sync_copy(src3d.at[s], src_v)
    pltpu.sync_copy(dst3d.at[s], dst_v)

    def pass_body(p, carry):
        q = PASSES * c + p
        _node_copy(s,
                   lambda off, n: xhs.at[q].at[pl.ds(off, n)],
                   lambda off, n: agg_sh.at[pl.ds(off, n)])
        plsc.subcore_barrier()

        def start_fetch(b):
            buf = lax.rem(b, 3)
            pltpu.async_copy(xhs.at[q].at[src_v.at[b]], rows_v.at[buf], gsem)
            pltpu.async_copy(ew3d16.at[ebase + b], w_v.at[buf], wsem)

        start_fetch(0)
        start_fetch(1)
        start_fetch(2)

        def body(b, carry2):
            buf = lax.rem(b, 3)
            pltpu.make_async_copy(
                xhs.at[q].at[src_v.at[b]], rows_v.at[buf], gsem).wait()
            pltpu.make_async_copy(
                ew3d16.at[ebase + b], w_v.at[buf], wsem).wait()

            @pl.when(b >= 3)
            def _():
                pltpu.make_async_copy(
                    sc_v.at[buf], agg_sh.at[dst_v.at[b - 3]], ssem).wait()

            for e in range(K):
                w = w_v[buf, e]
                for j in range(CHK // 16):
                    sl = pl.ds(j * 16, 16)
                    sc_v[buf, e, sl] = rows_v[buf, e, sl] * w

            pltpu.async_copy(sc_v.at[buf], agg_sh.at[dst_v.at[b]], ssem,
                             add=True)

            @pl.when(b + 3 < ROWS_C)
            def _():
                start_fetch(b + 3)

            return carry2

        lax.fori_loop(0, ROWS_C, body, 0)
        for tail in range(3):
            b = ROWS_C - 3 + tail
            pltpu.make_async_copy(
                sc_v.at[b % 3], agg_sh.at[dst_v.at[b]], ssem).wait()
        plsc.subcore_barrier()
        _node_copy(s,
                   lambda off, n: agg_sh.at[pl.ds(off, n)],
                   lambda off, n: aggp.at[q].at[pl.ds(off, n)])
        plsc.subcore_barrier()
        return carry

    lax.fori_loop(0, PASSES, pass_body, 0)


def _gate_body(dis_ref, aggp_ref, c_ref, w4_ref, b4_ref, h_out, c_out):
    dis = dis_ref[...]
    w4 = w4_ref[...]
    b4 = b4_ref[...]
    for b in range(2):
        agg = jnp.concatenate(
            [aggp_ref[4 * b + j] for j in range(4)], axis=1) * dis
        z = jnp.dot(agg, w4, preferred_element_type=jnp.float32) + b4
        i = jax.nn.sigmoid(z[:, :CH])
        f = jax.nn.sigmoid(z[:, CH:2 * CH])
        o = jax.nn.sigmoid(z[:, 2 * CH:3 * CH])
        g = jnp.tanh(z[:, 3 * CH:])
        cn = f * c_ref[b] + i * g
        h_out[b] = o * jnp.tanh(cn)
        c_out[b] = cn


_BG = 1000
_gates = pl.pallas_call(
    _gate_body,
    grid=(N // _BG,),
    in_specs=[
        pl.BlockSpec((_BG, 1), lambda i: (i, 0)),
        pl.BlockSpec((NQ, _BG, CHK), lambda i: (0, i, 0)),
        pl.BlockSpec((2, _BG, CH), lambda i: (0, i, 0)),
        pl.BlockSpec((2 * CH, 4 * CH), lambda i: (0, 0)),
        pl.BlockSpec((1, 4 * CH), lambda i: (0, 0)),
    ],
    out_specs=[
        pl.BlockSpec((2, _BG, CH), lambda i: (0, i, 0)),
        pl.BlockSpec((2, _BG, CH), lambda i: (0, i, 0)),
    ],
    out_shape=[
        jax.ShapeDtypeStruct((2, N, CH), jnp.float32),
        jax.ShapeDtypeStruct((2, N, CH), jnp.float32),
    ],
)


def kernel(x, h, c, edge_index, edge_weight, W_i, b_i, W_f, b_f, W_o, b_o, W_g, b_g):
    dst_a = edge_index[1].reshape(NC * NS, ROWS_A, K)
    ew_a = edge_weight.reshape(NC * NS, ROWS_A, K)
    src_c = edge_index[0].reshape(NS, ROWS_C, K)
    dst_c = edge_index[1].reshape(NS, ROWS_C, K)
    zeros_n = jnp.zeros((1, N), jnp.float32)
    degp = _deg_kernel(dst_a, ew_a, zeros_n)
    dis2d = _diskern(degp)
    xhs = _xscale(dis2d, x[0], h[0], x[1], h[1])
    ew16 = edge_weight.reshape(ROWS, K)
    ew3d16 = jnp.broadcast_to(ew16[:, :, None], (ROWS, K, 16))
    aggp = _prop_kernel(src_c, dst_c, ew3d16, xhs)
    w4 = jnp.concatenate([W_i, W_f, W_o, W_g], axis=1)
    b4 = jnp.concatenate([b_i, b_f, b_o, b_g]).reshape(1, 4 * CH)
    h_n, c_n = _gates(dis2d, aggp, c, w4, b4)
    return h_n, c_n


# speedup vs baseline: 30.1821x; 1.2180x over previous
import functools

import jax
import jax.numpy as jnp
from jax import lax
from jax.experimental import pallas as pl
from jax.experimental.pallas import tpu as pltpu
from jax.experimental.pallas import tpu_sc as plsc

N = 10000
E = 320000
CH = 128
CHK = 64
NQ = 8
K = 80
NC, NS = 2, 16
ROWS = E // K
ROWS_A = ROWS // (NC * NS)
ROWS_C = ROWS // NS
PASSES = NQ // NC

_MESH = plsc.VectorSubcoreMesh(
    core_axis_name="c", subcore_axis_name="s", num_cores=NC, num_subcores=NS)
_SC_PARAMS = pltpu.CompilerParams(use_tc_tiling_on_sc=False,
                                 needs_layout_passes=False)


@functools.partial(
    pl.kernel,
    out_type=jax.ShapeDtypeStruct((NC, 1, N), jnp.float32),
    mesh=_MESH,
    scratch_types=[
        pltpu.VMEM((ROWS_A, K), jnp.int32),
        pltpu.VMEM((ROWS_A, K), jnp.float32),
        pltpu.VMEM_SHARED((1, N), jnp.float32),
    ],
    compiler_params=_SC_PARAMS,
)
def _deg_kernel(dst3d, ew3d, zeros_n, degp, idx_v, ew_v, deg_sh):
    c = lax.axis_index("c")
    s = lax.axis_index("s")
    wid = s * NC + c

    @pl.when(s == 0)
    def _():
        pltpu.sync_copy(zeros_n, deg_sh)

    pltpu.sync_copy(dst3d.at[wid], idx_v)
    pltpu.sync_copy(ew3d.at[wid], ew_v)
    plsc.subcore_barrier()

    def body(b, carry):
        pltpu.sync_copy(ew_v.at[b], deg_sh.at[0].at[idx_v.at[b]], add=True)
        return carry

    lax.fori_loop(0, ROWS_A, body, 0)
    plsc.subcore_barrier()

    @pl.when(s == 0)
    def _():
        pltpu.sync_copy(deg_sh, degp.at[c])


def _dis_body(degp_ref, dis_ref):
    deg = degp_ref[0, 0, :] + degp_ref[1, 0, :] + 1.0
    dis = jnp.where(deg > 0, lax.rsqrt(jnp.maximum(deg, 1e-12)), 0.0)
    dis_ref[...] = dis[:, None]


_diskern = pl.pallas_call(
    _dis_body,
    in_specs=[pl.BlockSpec((2, 1, N), lambda: (0, 0, 0))],
    out_specs=pl.BlockSpec((N, 1), lambda: (0, 0)),
    out_shape=jax.ShapeDtypeStruct((N, 1), jnp.float32),
)


def _scale_body(dis_ref, x0, h0, x1, h1, out_ref):
    disc = dis_ref[...]
    srcs = (x0, h0, x1, h1)
    for qi in range(NQ):
        half = pl.ds((qi % 2) * CHK, CHK)
        out_ref[qi] = srcs[qi // 2][:, half] * disc


_BN = 2000
_xscale = pl.pallas_call(
    _scale_body,
    grid=(N // _BN,),
    in_specs=[
        pl.BlockSpec((_BN, 1), lambda i: (i, 0)),
        pl.BlockSpec((_BN, CH), lambda i: (i, 0)),
        pl.BlockSpec((_BN, CH), lambda i: (i, 0)),
        pl.BlockSpec((_BN, CH), lambda i: (i, 0)),
        pl.BlockSpec((_BN, CH), lambda i: (i, 0)),
    ],
    out_specs=pl.BlockSpec((NQ, _BN, CHK), lambda i: (0, i, 0)),
    out_shape=jax.ShapeDtypeStruct((NQ, N, CHK), jnp.float32),
)


_SPLIT = 640
_LAST = N - (NS - 1) * _SPLIT


def _node_copy(s, mk_src, mk_dst):
    @pl.when(s < NS - 1)
    def _():
        off = pl.multiple_of(s * _SPLIT, _SPLIT)
        pltpu.sync_copy(mk_src(off, _SPLIT), mk_dst(off, _SPLIT))

    @pl.when(s == NS - 1)
    def _():
        off = (NS - 1) * _SPLIT
        pltpu.sync_copy(mk_src(off, _LAST), mk_dst(off, _LAST))


@functools.partial(
    pl.kernel,
    out_type=jax.ShapeDtypeStruct((NQ, N, CHK), jnp.float32),
    mesh=_MESH,
    scratch_types=[
        pltpu.VMEM((ROWS_C, K), jnp.int32),
        pltpu.VMEM((ROWS_C, K), jnp.int32),
        pltpu.VMEM((ROWS_C, K), jnp.float32),
        pltpu.VMEM((3, K, CHK), jnp.float32),
        pltpu.VMEM((3, K, CHK), jnp.float32),
        pltpu.VMEM_SHARED((N, CHK), jnp.float32),
        pltpu.SemaphoreType.DMA,
        pltpu.SemaphoreType.DMA,
    ],
    compiler_params=_SC_PARAMS,
)
def _prop_kernel(src3d, dst3d, ew3d, xhs, aggp,
                 src_v, dst_v, ew_v, rows_v, sc_v, agg_sh,
                 gsem, ssem):
    c = lax.axis_index("c")
    s = lax.axis_index("s")
    pltpu.sync_copy(src3d.at[s], src_v)
    pltpu.sync_copy(dst3d.at[s], dst_v)
    pltpu.sync_copy(ew3d.at[s], ew_v)

    def pass_body(p, carry):
        q = PASSES * c + p
        _node_copy(s,
                   lambda off, n: xhs.at[q].at[pl.ds(off, n)],
                   lambda off, n: agg_sh.at[pl.ds(off, n)])
        plsc.subcore_barrier()

        def start_fetch(b):
            buf = lax.rem(b, 3)
            pltpu.async_copy(xhs.at[q].at[src_v.at[b]], rows_v.at[buf], gsem)

        start_fetch(0)
        start_fetch(1)
        start_fetch(2)

        def body(b, carry2):
            buf = lax.rem(b, 3)
            pltpu.make_async_copy(
                xhs.at[q].at[src_v.at[b]], rows_v.at[buf], gsem).wait()

            @pl.when(b >= 3)
            def _():
                pltpu.make_async_copy(
                    sc_v.at[buf], agg_sh.at[dst_v.at[b - 3]], ssem).wait()

            for e in range(K):
                w = plsc.load_gather(
                    ew_v, [jnp.full((16,), b, jnp.int32),
                           jnp.full((16,), e, jnp.int32)])
                for j in range(CHK // 16):
                    sl = pl.ds(j * 16, 16)
                    sc_v[buf, e, sl] = rows_v[buf, e, sl] * w

            pltpu.async_copy(sc_v.at[buf], agg_sh.at[dst_v.at[b]], ssem,
                             add=True)

            @pl.when(b + 3 < ROWS_C)
            def _():
                start_fetch(b + 3)

            return carry2

        lax.fori_loop(0, ROWS_C, body, 0)
        for tail in range(3):
            b = ROWS_C - 3 + tail
            pltpu.make_async_copy(
                sc_v.at[b % 3], agg_sh.at[dst_v.at[b]], ssem).wait()
        plsc.subcore_barrier()
        _node_copy(s,
                   lambda off, n: agg_sh.at[pl.ds(off, n)],
                   lambda off, n: aggp.at[q].at[pl.ds(off, n)])
        plsc.subcore_barrier()
        return carry

    lax.fori_loop(0, PASSES, pass_body, 0)


def _gate_body(dis_ref, aggp_ref, c_ref, w4_ref, b4_ref, h_out, c_out):
    dis = dis_ref[...]
    w4 = w4_ref[...]
    b4 = b4_ref[...]
    for b in range(2):
        agg = jnp.concatenate(
            [aggp_ref[4 * b + j] for j in range(4)], axis=1) * dis
        z = jnp.dot(agg, w4, preferred_element_type=jnp.float32) + b4
        i = jax.nn.sigmoid(z[:, :CH])
        f = jax.nn.sigmoid(z[:, CH:2 * CH])
        o = jax.nn.sigmoid(z[:, 2 * CH:3 * CH])
        g = jnp.tanh(z[:, 3 * CH:])
        cn = f * c_ref[b] + i * g
        h_out[b] = o * jnp.tanh(cn)
        c_out[b] = cn


_BG = 1000
_gates = pl.pallas_call(
    _gate_body,
    grid=(N // _BG,),
    in_specs=[
        pl.BlockSpec((_BG, 1), lambda i: (i, 0)),
        pl.BlockSpec((NQ, _BG, CHK), lambda i: (0, i, 0)),
        pl.BlockSpec((2, _BG, CH), lambda i: (0, i, 0)),
        pl.BlockSpec((2 * CH, 4 * CH), lambda i: (0, 0)),
        pl.BlockSpec((1, 4 * CH), lambda i: (0, 0)),
    ],
    out_specs=[
        pl.BlockSpec((2, _BG, CH), lambda i: (0, i, 0)),
        pl.BlockSpec((2, _BG, CH), lambda i: (0, i, 0)),
    ],
    out_shape=[
        jax.ShapeDtypeStruct((2, N, CH), jnp.float32),
        jax.ShapeDtypeStruct((2, N, CH), jnp.float32),
    ],
)


def kernel(x, h, c, edge_index, edge_weight, W_i, b_i, W_f, b_f, W_o, b_o, W_g, b_g):
    dst_a = edge_index[1].reshape(NC * NS, ROWS_A, K)
    ew_a = edge_weight.reshape(NC * NS, ROWS_A, K)
    src_c = edge_index[0].reshape(NS, ROWS_C, K)
    dst_c = edge_index[1].reshape(NS, ROWS_C, K)
    zeros_n = jnp.zeros((1, N), jnp.float32)
    degp = _deg_kernel(dst_a, ew_a, zeros_n)
    dis2d = _diskern(degp)
    xhs = _xscale(dis2d, x[0], h[0], x[1], h[1])
    ew_c = edge_weight.reshape(NS, ROWS_C, K)
    aggp = _prop_kernel(src_c, dst_c, ew_c, xhs)
    w4 = jnp.concatenate([W_i, W_f, W_o, W_g], axis=1)
    b4 = jnp.concatenate([b_i, b_f, b_o, b_g]).reshape(1, 4 * CH)
    h_n, c_n = _gates(dis2d, aggp, c, w4, b4)
    return h_n, c_n


# speedup vs baseline: 40.3515x; 1.3369x over previous
import functools

import jax
import jax.numpy as jnp
from jax import lax
from jax.experimental import pallas as pl
from jax.experimental.pallas import tpu as pltpu
from jax.experimental.pallas import tpu_sc as plsc

N = 10000
E = 320000
CH = 128
GW = 64
NQ = 4
K = 80
NC, NS = 2, 16
ROWS = E // K
ROWS_A = ROWS // (NC * NS)
ROWS_C = ROWS // NS
PASSES = NQ // NC

_MESH = plsc.VectorSubcoreMesh(
    core_axis_name="c", subcore_axis_name="s", num_cores=NC, num_subcores=NS)
_SC_PARAMS = pltpu.CompilerParams(use_tc_tiling_on_sc=False,
                                 needs_layout_passes=False)


@functools.partial(
    pl.kernel,
    out_type=jax.ShapeDtypeStruct((NC, 1, N), jnp.float32),
    mesh=_MESH,
    scratch_types=[
        pltpu.VMEM((ROWS_A, K), jnp.int32),
        pltpu.VMEM((ROWS_A, K), jnp.float32),
        pltpu.VMEM_SHARED((1, N), jnp.float32),
    ],
    compiler_params=_SC_PARAMS,
)
def _deg_kernel(dst3d, ew3d, zeros_n, degp, idx_v, ew_v, deg_sh):
    c = lax.axis_index("c")
    s = lax.axis_index("s")
    wid = s * NC + c

    @pl.when(s == 0)
    def _():
        pltpu.sync_copy(zeros_n, deg_sh)

    pltpu.sync_copy(dst3d.at[wid], idx_v)
    pltpu.sync_copy(ew3d.at[wid], ew_v)
    plsc.subcore_barrier()

    def body(b, carry):
        pltpu.sync_copy(ew_v.at[b], deg_sh.at[0].at[idx_v.at[b]], add=True)
        return carry

    lax.fori_loop(0, ROWS_A, body, 0)
    plsc.subcore_barrier()

    @pl.when(s == 0)
    def _():
        pltpu.sync_copy(deg_sh, degp.at[c])


def _dis_body(degp_ref, dis_ref):
    deg = degp_ref[0, 0, :] + degp_ref[1, 0, :] + 1.0
    dis = jnp.where(deg > 0, lax.rsqrt(jnp.maximum(deg, 1e-12)), 0.0)
    dis_ref[...] = dis[:, None]


_diskern = pl.pallas_call(
    _dis_body,
    in_specs=[pl.BlockSpec((2, 1, N), lambda: (0, 0, 0))],
    out_specs=pl.BlockSpec((N, 1), lambda: (0, 0)),
    out_shape=jax.ShapeDtypeStruct((N, 1), jnp.float32),
)


def _scale_body(dis_ref, x0, h0, x1, h1, bf_ref):
    disc = dis_ref[...]
    srcs = (x0, h0, x1, h1)
    for qi in range(NQ):
        bf_ref[qi] = (srcs[qi][...] * disc).astype(jnp.bfloat16)


_BN = 2000
_xscale = pl.pallas_call(
    _scale_body,
    grid=(N // _BN,),
    in_specs=[
        pl.BlockSpec((_BN, 1), lambda i: (i, 0)),
        pl.BlockSpec((_BN, CH), lambda i: (i, 0)),
        pl.BlockSpec((_BN, CH), lambda i: (i, 0)),
        pl.BlockSpec((_BN, CH), lambda i: (i, 0)),
        pl.BlockSpec((_BN, CH), lambda i: (i, 0)),
    ],
    out_specs=pl.BlockSpec((NQ, _BN, CH), lambda i: (0, i, 0)),
    out_shape=jax.ShapeDtypeStruct((NQ, N, CH), jnp.bfloat16),
)


_SPLIT = 640
_LAST = N - (NS - 1) * _SPLIT


def _node_copy(s, mk_src, mk_dst):
    @pl.when(s < NS - 1)
    def _():
        off = pl.multiple_of(s * _SPLIT, _SPLIT)
        pltpu.sync_copy(mk_src(off, _SPLIT), mk_dst(off, _SPLIT))

    @pl.when(s == NS - 1)
    def _():
        off = (NS - 1) * _SPLIT
        pltpu.sync_copy(mk_src(off, _LAST), mk_dst(off, _LAST))


@functools.partial(
    pl.kernel,
    out_type=jax.ShapeDtypeStruct((NQ, N, CH), jnp.bfloat16),
    mesh=_MESH,
    scratch_types=[
        pltpu.VMEM((ROWS_C, K), jnp.int32),
        pltpu.VMEM((ROWS_C, K), jnp.int32),
        pltpu.VMEM((ROWS_C, K), jnp.float32),
        pltpu.VMEM((3, K, GW), jnp.int32),
        pltpu.VMEM((3, K, CH), jnp.bfloat16),
        pltpu.VMEM_SHARED((N, CH), jnp.bfloat16),
        pltpu.SemaphoreType.DMA,
        pltpu.SemaphoreType.DMA,
    ],
    compiler_params=_SC_PARAMS,
)
def _prop_kernel(src3d, dst3d, ew3d, xhs_pk, xhs_bf, aggp,
                 src_v, dst_v, ew_v, rows_v, sc_v, agg_sh,
                 gsem, ssem):
    c = lax.axis_index("c")
    s = lax.axis_index("s")
    pltpu.sync_copy(src3d.at[s], src_v)
    pltpu.sync_copy(dst3d.at[s], dst_v)
    pltpu.sync_copy(ew3d.at[s], ew_v)

    def pass_body(p, carry):
        q = PASSES * c + p
        _node_copy(s,
                   lambda off, n: xhs_bf.at[q].at[pl.ds(off, n)],
                   lambda off, n: agg_sh.at[pl.ds(off, n)])
        plsc.subcore_barrier()

        def start_fetch(b):
            buf = lax.rem(b, 3)
            pltpu.async_copy(xhs_pk.at[q].at[src_v.at[b]], rows_v.at[buf], gsem)

        start_fetch(0)
        start_fetch(1)
        start_fetch(2)

        def body(b, carry2):
            buf = lax.rem(b, 3)
            pltpu.make_async_copy(
                xhs_pk.at[q].at[src_v.at[b]], rows_v.at[buf], gsem).wait()

            @pl.when(b >= 3)
            def _():
                pltpu.make_async_copy(
                    sc_v.at[buf], agg_sh.at[dst_v.at[b - 3]], ssem).wait()

            for e in range(K):
                w32 = plsc.load_gather(
                    ew_v, [jnp.full((16,), b, jnp.int32),
                           jnp.full((16,), e, jnp.int32)])
                wbf = plsc.pack(w32, w32, format=plsc.PackFormat.INTERLEAVED)
                for j in range(GW // 16):
                    packed = rows_v[buf, e, pl.ds(j * 16, 16)]
                    vals = plsc.bitcast(packed, jnp.bfloat16)
                    sc_v[buf, e, pl.ds(j * 32, 32)] = vals * wbf

            pltpu.async_copy(sc_v.at[buf], agg_sh.at[dst_v.at[b]], ssem,
                             add=True)

            @pl.when(b + 3 < ROWS_C)
            def _():
                start_fetch(b + 3)

            return carry2

        lax.fori_loop(0, ROWS_C, body, 0)
        for tail in range(3):
            b = ROWS_C - 3 + tail
            pltpu.make_async_copy(
                sc_v.at[b % 3], agg_sh.at[dst_v.at[b]], ssem).wait()
        plsc.subcore_barrier()
        _node_copy(s,
                   lambda off, n: agg_sh.at[pl.ds(off, n)],
                   lambda off, n: aggp.at[q].at[pl.ds(off, n)])
        plsc.subcore_barrier()
        return carry

    lax.fori_loop(0, PASSES, pass_body, 0)


def _gate_body(dis_ref, aggp_ref, c_ref, w4_ref, b4_ref, h_out, c_out):
    dis = dis_ref[...]
    w4 = w4_ref[...]
    b4 = b4_ref[...]
    for b in range(2):
        agg = jnp.concatenate(
            [aggp_ref[2 * b + j].astype(jnp.float32) for j in range(2)],
            axis=1) * dis
        z = jnp.dot(agg, w4, preferred_element_type=jnp.float32) + b4
        i = jax.nn.sigmoid(z[:, :CH])
        f = jax.nn.sigmoid(z[:, CH:2 * CH])
        o = jax.nn.sigmoid(z[:, 2 * CH:3 * CH])
        g = jnp.tanh(z[:, 3 * CH:])
        cn = f * c_ref[b] + i * g
        h_out[b] = o * jnp.tanh(cn)
        c_out[b] = cn


_BG = 1000
_gates = pl.pallas_call(
    _gate_body,
    grid=(N // _BG,),
    in_specs=[
        pl.BlockSpec((_BG, 1), lambda i: (i, 0)),
        pl.BlockSpec((NQ, _BG, CH), lambda i: (0, i, 0)),
        pl.BlockSpec((2, _BG, CH), lambda i: (0, i, 0)),
        pl.BlockSpec((2 * CH, 4 * CH), lambda i: (0, 0)),
        pl.BlockSpec((1, 4 * CH), lambda i: (0, 0)),
    ],
    out_specs=[
        pl.BlockSpec((2, _BG, CH), lambda i: (0, i, 0)),
        pl.BlockSpec((2, _BG, CH), lambda i: (0, i, 0)),
    ],
    out_shape=[
        jax.ShapeDtypeStruct((2, N, CH), jnp.float32),
        jax.ShapeDtypeStruct((2, N, CH), jnp.float32),
    ],
)


def kernel(x, h, c, edge_index, edge_weight, W_i, b_i, W_f, b_f, W_o, b_o, W_g, b_g):
    dst_a = edge_index[1].reshape(NC * NS, ROWS_A, K)
    ew_a = edge_weight.reshape(NC * NS, ROWS_A, K)
    src_c = edge_index[0].reshape(NS, ROWS_C, K)
    dst_c = edge_index[1].reshape(NS, ROWS_C, K)
    zeros_n = jnp.zeros((1, N), jnp.float32)
    degp = _deg_kernel(dst_a, ew_a, zeros_n)
    dis2d = _diskern(degp)
    xhs_bf = _xscale(dis2d, x[0], h[0], x[1], h[1])
    xhs_pk = lax.bitcast_convert_type(
        xhs_bf.reshape(NQ, N, GW, 2), jnp.int32)
    ew_c = edge_weight.reshape(NS, ROWS_C, K)
    aggp = _prop_kernel(src_c, dst_c, ew_c, xhs_pk, xhs_bf)
    w4 = jnp.concatenate([W_i, W_f, W_o, W_g], axis=1)
    b4 = jnp.concatenate([b_i, b_f, b_o, b_g]).reshape(1, 4 * CH)
    h_n, c_n = _gates(dis2d, aggp, c, w4, b4)
    return h_n, c_n
